# baseline (speedup 1.0000x reference)
import functools

import jax
import jax.numpy as jnp
from jax import lax
from jax.experimental import pallas as pl
from jax.experimental.pallas import tpu as pltpu
from jax.experimental.pallas import tpu_sc as plsc

VOCAB = 1000000
D = 64
SEQ = 2048
BATCH = 16

NC = 2
NS = 16
NW = NC * NS
TOTAL = BATCH * SEQ
ROWS_PER_W = TOTAL // NW
CHUNK = 128
NCHUNK = ROWS_PER_W // CHUNK
SCALE = 8.0


def _body(ids_hbm, tok_hbm, pos_hbm, out_hbm, idx_v, rows_v, pos_v, sem):
    c_ax = lax.axis_index("c")
    s_ax = lax.axis_index("s")
    wid = s_ax * NC + c_ax
    base = wid * ROWS_PER_W
    pos_base = (wid % 2) * ROWS_PER_W

    pltpu.sync_copy(ids_hbm.at[pl.ds(wid * NCHUNK, NCHUNK)], idx_v)

    def chunk_body(c, _):
        pltpu.async_copy(tok_hbm.at[idx_v.at[c]], rows_v, sem).wait()
        pltpu.sync_copy(pos_hbm.at[pl.ds(pos_base + c * CHUNK, CHUNK)], pos_v)

        def row_body(j, _):
            for k in range(D // 16):
                sl = pl.ds(k * 16, 16)
                rows_v[j, sl] = (rows_v[j, sl] + pos_v[j, sl]) * SCALE
            return 0

        lax.fori_loop(0, CHUNK, row_body, 0)
        pltpu.sync_copy(rows_v, out_hbm.at[pl.ds(base + c * CHUNK, CHUNK)])
        return 0

    lax.fori_loop(0, NCHUNK, chunk_body, 0)


@jax.jit
def _emb(ids2d, tok, pos):
    mesh = plsc.VectorSubcoreMesh(core_axis_name="c", subcore_axis_name="s")
    f = pl.kernel(
        _body,
        out_type=jax.ShapeDtypeStruct((TOTAL, D), jnp.float32),
        mesh=mesh,
        scratch_types=[
            pltpu.VMEM((NCHUNK, CHUNK), jnp.int32),
            pltpu.VMEM((CHUNK, D), jnp.float32),
            pltpu.VMEM((CHUNK, D), jnp.float32),
            pltpu.SemaphoreType.DMA,
        ],
        compiler_params=pltpu.CompilerParams(use_tc_tiling_on_sc=False),
    )
    return f(ids2d, tok, pos)


def kernel(token_ids, token_table, pos_table):
    ids2d = jnp.asarray(token_ids, jnp.int32).reshape(NW * NCHUNK, CHUNK)
    out = _emb(ids2d, token_table, pos_table)
    return out.reshape(BATCH, SEQ, D)


# speedup vs baseline: 1.0107x; 1.0107x over previous
import jax
import jax.numpy as jnp
from jax import lax
from jax.experimental import pallas as pl
from jax.experimental.pallas import tpu as pltpu
from jax.experimental.pallas import tpu_sc as plsc

VOCAB = 1000000
D = 64
SEQ = 2048
BATCH = 16

NC = 2
NS = 16
NW = NC * NS
TOTAL = BATCH * SEQ
ROWS_PER_W = TOTAL // NW
CHUNK = 128
NCHUNK = ROWS_PER_W // CHUNK
SCALE = 8.0


def _body(ids_hbm, tok_hbm, pos_hbm, out_hbm, idx_v, rows_v, pos_v,
          gsem0, gsem1, psem0, psem1, ssem0, ssem1):
    c_ax = lax.axis_index("c")
    s_ax = lax.axis_index("s")
    wid = s_ax * NC + c_ax
    base = wid * ROWS_PER_W
    pos_base = (wid % 2) * ROWS_PER_W

    gsem = (gsem0, gsem1)
    psem = (psem0, psem1)
    ssem = (ssem0, ssem1)

    pltpu.sync_copy(ids_hbm.at[pl.ds(wid * NCHUNK, NCHUNK)], idx_v)

    def start_fetch(c, b):
        gd = pltpu.async_copy(tok_hbm.at[idx_v.at[c]], rows_v.at[b], gsem[b])
        pd = pltpu.async_copy(
            pos_hbm.at[pl.ds(pos_base + c * CHUNK, CHUNK)], pos_v.at[b],
            psem[b])
        return gd, pd

    def compute(b):
        @plsc.parallel_loop(0, CHUNK, step=1, unroll=4)
        def _(j):
            for k in range(D // 16):
                sl = pl.ds(k * 16, 16)
                rows_v[b, j, sl] = (rows_v[b, j, sl] + pos_v[b, j, sl]) * SCALE

    fetch = [None] * NCHUNK
    store = [None] * NCHUNK
    fetch[0] = start_fetch(0, 0)
    for c in range(NCHUNK):
        b = c % 2
        if c + 1 < NCHUNK:
            if c >= 1:
                store[c - 1].wait()
            fetch[c + 1] = start_fetch(c + 1, (c + 1) % 2)
        gd, pd = fetch[c]
        gd.wait()
        pd.wait()
        compute(b)
        store[c] = pltpu.async_copy(
            rows_v.at[b], out_hbm.at[pl.ds(base + c * CHUNK, CHUNK)], ssem[b])
    store[NCHUNK - 2].wait()
    store[NCHUNK - 1].wait()


@jax.jit
def _emb(ids2d, tok, pos):
    mesh = plsc.VectorSubcoreMesh(core_axis_name="c", subcore_axis_name="s")
    f = pl.kernel(
        _body,
        out_type=jax.ShapeDtypeStruct((TOTAL, D), jnp.float32),
        mesh=mesh,
        scratch_types=[
            pltpu.VMEM((NCHUNK, CHUNK), jnp.int32),
            pltpu.VMEM((2, CHUNK, D), jnp.float32),
            pltpu.VMEM((2, CHUNK, D), jnp.float32),
            pltpu.SemaphoreType.DMA,
            pltpu.SemaphoreType.DMA,
            pltpu.SemaphoreType.DMA,
            pltpu.SemaphoreType.DMA,
            pltpu.SemaphoreType.DMA,
            pltpu.SemaphoreType.DMA,
        ],
        compiler_params=pltpu.CompilerParams(use_tc_tiling_on_sc=False),
    )
    return f(ids2d, tok, pos)


def kernel(token_ids, token_table, pos_table):
    ids2d = jnp.asarray(token_ids, jnp.int32).reshape(NW * NCHUNK, CHUNK)
    out = _emb(ids2d, token_table, pos_table)
    return out.reshape(BATCH, SEQ, D)


# speedup vs baseline: 1.6751x; 1.6574x over previous
import jax
import jax.numpy as jnp
from jax import lax
from jax.experimental import pallas as pl
from jax.experimental.pallas import tpu as pltpu
from jax.experimental.pallas import tpu_sc as plsc

VOCAB = 1000000
D = 64
SEQ = 2048
BATCH = 16

NC = 2
NS = 16
NW = NC * NS
TOTAL = BATCH * SEQ
ROWS_PER_W = TOTAL // NW
CB = 128
NCHUNK = ROWS_PER_W // CB
SCALE = 8.0


def _body(ids_hbm, tok_hbm, pos_hbm, out_hbm,
          ids_s, rows_v, pos_v,
          gsem0, gsem1, psem0, psem1, ssem0, ssem1):
    c_ax = lax.axis_index("c")
    s_ax = lax.axis_index("s")
    wid = s_ax * NC + c_ax
    base = wid * ROWS_PER_W
    pos_base = (wid % 2) * ROWS_PER_W

    gsem = (gsem0, gsem1)
    psem = (psem0, psem1)
    ssem = (ssem0, ssem1)

    pltpu.sync_copy(ids_hbm.at[pl.ds(base, ROWS_PER_W)], ids_s)

    def start_fetch(c, b):
        def fire(g, _):
            tvec = ids_s[pl.ds(c * CB + g * 16, 16)]
            for l in range(16):
                tid = tvec[l]
                pltpu.async_copy(
                    tok_hbm.at[pl.ds(tid, 1)],
                    rows_v.at[b, pl.ds(g * 16 + l, 1)], gsem[b])
            return 0

        lax.fori_loop(0, CB // 16, fire, 0)
        pd = pltpu.async_copy(
            pos_hbm.at[pl.ds(pos_base + c * CB, CB)], pos_v.at[b], psem[b])
        return pd

    def drain_fetch(b):
        pltpu.make_async_copy(
            tok_hbm.at[pl.ds(0, CB)], rows_v.at[b], gsem[b]).wait()

    def compute(b):
        @plsc.parallel_loop(0, CB, step=1, unroll=4)
        def _(j):
            for k in range(D // 16):
                sl = pl.ds(k * 16, 16)
                rows_v[b, j, sl] = (rows_v[b, j, sl] + pos_v[b, j, sl]) * SCALE

    fetch = [None] * NCHUNK
    store = [None] * NCHUNK
    fetch[0] = start_fetch(0, 0)
    for c in range(NCHUNK):
        b = c % 2
        if c + 1 < NCHUNK:
            if c >= 1:
                store[c - 1].wait()
            fetch[c + 1] = start_fetch(c + 1, (c + 1) % 2)
        drain_fetch(b)
        fetch[c].wait()
        compute(b)
        store[c] = pltpu.async_copy(
            rows_v.at[b], out_hbm.at[pl.ds(base + c * CB, CB)], ssem[b])
    store[NCHUNK - 2].wait()
    store[NCHUNK - 1].wait()


@jax.jit
def _emb(ids, tok, pos):
    mesh = plsc.VectorSubcoreMesh(core_axis_name="c", subcore_axis_name="s")
    f = pl.kernel(
        _body,
        out_type=jax.ShapeDtypeStruct((TOTAL, D), jnp.float32),
        mesh=mesh,
        scratch_types=[
            pltpu.VMEM((ROWS_PER_W,), jnp.int32),
            pltpu.VMEM((2, CB, D), jnp.float32),
            pltpu.VMEM((2, CB, D), jnp.float32),
            pltpu.SemaphoreType.DMA,
            pltpu.SemaphoreType.DMA,
            pltpu.SemaphoreType.DMA,
            pltpu.SemaphoreType.DMA,
            pltpu.SemaphoreType.DMA,
            pltpu.SemaphoreType.DMA,
        ],
    )
    return f(ids, tok, pos)


def kernel(token_ids, token_table, pos_table):
    ids = jnp.asarray(token_ids, jnp.int32).reshape(TOTAL)
    out = _emb(ids, token_table, pos_table)
    return out.reshape(BATCH, SEQ, D)


# speedup vs baseline: 1.6848x; 1.0057x over previous
import jax
import jax.numpy as jnp
from jax import lax
from jax.experimental import pallas as pl
from jax.experimental.pallas import tpu as pltpu
from jax.experimental.pallas import tpu_sc as plsc

VOCAB = 1000000
D = 64
SEQ = 2048
BATCH = 16

NC = 2
NS = 16
NW = NC * NS
TOTAL = BATCH * SEQ
ROWS_PER_W = TOTAL // NW
HALF = SEQ // 2
CB = 128
NCHUNK = ROWS_PER_W // CB
SCALE = 8.0


def _body(ids_hbm, tok_hbm, pos_hbm, out_hbm,
          ids_v, pos_v, rows_v,
          gsem0, gsem1, psem0, psem1, ssem0, ssem1):
    c_ax = lax.axis_index("c")
    s_ax = lax.axis_index("s")
    wid = s_ax * NC + c_ax
    brow = wid // 2
    half = wid % 2
    base = wid * ROWS_PER_W
    pos_base = half * HALF

    gsem = (gsem0, gsem1)
    psem = (psem0, psem1)
    ssem = (ssem0, ssem1)

    pltpu.sync_copy(ids_hbm.at[brow, pl.ds(pos_base, ROWS_PER_W)], ids_v)

    def start_fetch(c, b):
        pltpu.async_copy(
            pos_hbm.at[pl.ds(pos_base + c * CB, CB)], pos_v.at[b], psem[b])

        def fire(g, _):
            tvec = ids_v[pl.ds(c * CB + g * 16, 16)]
            for l in range(16):
                tid = tvec[l]
                pltpu.async_copy(
                    tok_hbm.at[pl.ds(tid, 1)],
                    rows_v.at[b, pl.ds(g * 16 + l, 1)], gsem[b])
            return 0

        lax.fori_loop(0, CB // 16, fire, 0)

    def drain_fetch(b):
        pltpu.make_async_copy(
            tok_hbm.at[pl.ds(0, CB)], rows_v.at[b], gsem[b]).wait()
        pltpu.make_async_copy(
            pos_hbm.at[pl.ds(0, CB)], pos_v.at[b], psem[b]).wait()

    def compute(c, b):
        @plsc.parallel_loop(0, CB, step=1, unroll=4)
        def _(j):
            for k in range(D // 16):
                sl = pl.ds(k * 16, 16)
                rows_v[b, j, sl] = (
                    rows_v[b, j, sl] + pos_v[b, j, sl]) * SCALE

    store = [None] * NCHUNK
    start_fetch(0, 0)
    for c in range(NCHUNK):
        b = c % 2
        if c + 1 < NCHUNK:
            if c >= 1:
                store[c - 1].wait()
            start_fetch(c + 1, (c + 1) % 2)
        drain_fetch(b)
        compute(c, b)
        store[c] = pltpu.async_copy(
            rows_v.at[b], out_hbm.at[pl.ds(base + c * CB, CB)], ssem[b])
    store[NCHUNK - 2].wait()
    store[NCHUNK - 1].wait()


@jax.jit
def _emb(ids, tok, pos):
    mesh = plsc.VectorSubcoreMesh(core_axis_name="c", subcore_axis_name="s")
    f = pl.kernel(
        _body,
        out_type=jax.ShapeDtypeStruct((TOTAL, D), jnp.float32),
        mesh=mesh,
        scratch_types=[
            pltpu.VMEM((ROWS_PER_W,), jnp.int32),
            pltpu.VMEM((2, CB, D), jnp.float32),
            pltpu.VMEM((2, CB, D), jnp.float32),
            pltpu.SemaphoreType.DMA,
            pltpu.SemaphoreType.DMA,
            pltpu.SemaphoreType.DMA,
            pltpu.SemaphoreType.DMA,
            pltpu.SemaphoreType.DMA,
            pltpu.SemaphoreType.DMA,
        ],
    )
    return f(ids, tok, pos)


def kernel(token_ids, token_table, pos_table):
    ids = jnp.asarray(token_ids, jnp.int32)
    out = _emb(ids, token_table, pos_table)
    return out.reshape(BATCH, SEQ, D)
